# Initial kernel scaffold; baseline (speedup 1.0000x reference)
import jax
import jax.numpy as jnp
from jax.experimental import pallas as pl


def kernel(image, lut):
    raise NotImplementedError("write your pallas kernel here")


# speedup vs baseline: 901.6604x; 901.6604x over previous
import functools

import jax
import jax.numpy as jnp
from jax import lax
from jax.experimental import pallas as pl
from jax.experimental.pallas import tpu as pltpu
from jax.experimental.pallas import tpu_sc as plsc

NC = 2
NS = 16
NW = NC * NS
LANES = 16

CHUNK = 2048


@functools.lru_cache(maxsize=None)
def _make_sc_kernel(B: int, HW: int, L: int):
    lut_words = L * L * L * 3
    lut_pad = -(-lut_words // 16) * 16
    DR = 3 * L * L
    DG = 3 * L
    DB = 3
    pw = (B * HW) // (B * NW)
    assert HW % NW == 0 and pw % CHUNK == 0 and CHUNK % LANES == 0
    nchunk = pw // CHUNK
    nvec = CHUNK // LANES

    mesh = plsc.VectorSubcoreMesh(core_axis_name="c", subcore_axis_name="s")

    @functools.partial(
        pl.kernel,
        mesh=mesh,
        compiler_params=pltpu.CompilerParams(needs_layout_passes=False),
        out_type=jax.ShapeDtypeStruct((B * 3 * HW,), jnp.float32),
        scratch_types=[
            pltpu.VMEM((lut_pad,), jnp.float32),
            pltpu.VMEM((CHUNK,), jnp.float32),
            pltpu.VMEM((CHUNK,), jnp.float32),
            pltpu.VMEM((CHUNK,), jnp.float32),
            pltpu.VMEM((CHUNK,), jnp.float32),
            pltpu.VMEM((CHUNK,), jnp.float32),
            pltpu.VMEM((CHUNK,), jnp.float32),
            pltpu.SemaphoreType.DMA,
        ],
    )
    def sc_kernel(img, lutf, out, lut_v, in_r, in_g, in_b, o_r, o_g, o_b, sem):
        cid = lax.axis_index("c")
        sid = lax.axis_index("s")
        wid = sid * NC + cid
        pltpu.sync_copy(lutf, lut_v)

        def vec_body(i, _):
            s = i * LANES

            def channel(ref):
                x = ref[pl.ds(s, LANES)] * jnp.float32(L - 1)
                ix = x.astype(jnp.int32)
                fr = x - ix.astype(jnp.float32)
                i0 = jnp.minimum(ix, L - 2)
                return i0, fr

            ir, fr = channel(in_r)
            ig, fg = channel(in_g)
            ib, fb = channel(in_b)
            base = ((ir * L + ig) * L + ib) * 3

            def gat(off):
                return plsc.load_gather(lut_v, [base + off])

            def lerp(a, b, t):
                return a + (b - a) * t

            res = []
            for ch in range(3):
                c000 = gat(ch)
                c100 = gat(DR + ch)
                c010 = gat(DG + ch)
                c110 = gat(DR + DG + ch)
                c001 = gat(DB + ch)
                c101 = gat(DR + DB + ch)
                c011 = gat(DG + DB + ch)
                c111 = gat(DR + DG + DB + ch)
                c00 = lerp(c000, c100, fr)
                c01 = lerp(c001, c101, fr)
                c10 = lerp(c010, c110, fr)
                c11 = lerp(c011, c111, fr)
                c0 = lerp(c00, c10, fg)
                c1 = lerp(c01, c11, fg)
                res.append(lerp(c0, c1, fb))
            o_r[pl.ds(s, LANES)] = res[0]
            o_g[pl.ds(s, LANES)] = res[1]
            o_b[pl.ds(s, LANES)] = res[2]
            return 0

        for b in range(B):
            def chunk_body(k, _, b=b):
                off = wid * pw + k * CHUNK
                r0 = (b * 3 + 0) * HW + off
                g0 = (b * 3 + 1) * HW + off
                b0 = (b * 3 + 2) * HW + off
                cr = pltpu.async_copy(img.at[pl.ds(r0, CHUNK)], in_r, sem)
                cg = pltpu.async_copy(img.at[pl.ds(g0, CHUNK)], in_g, sem)
                cb = pltpu.async_copy(img.at[pl.ds(b0, CHUNK)], in_b, sem)
                cr.wait()
                cg.wait()
                cb.wait()
                lax.fori_loop(0, nvec, vec_body, 0)
                wr = pltpu.async_copy(o_r, out.at[pl.ds(r0, CHUNK)], sem)
                wg = pltpu.async_copy(o_g, out.at[pl.ds(g0, CHUNK)], sem)
                wb = pltpu.async_copy(o_b, out.at[pl.ds(b0, CHUNK)], sem)
                wr.wait()
                wg.wait()
                wb.wait()
                return 0

            lax.fori_loop(0, nchunk, chunk_body, 0)

    return sc_kernel


def kernel(image, lut):
    B, C, H, W = image.shape
    L = lut.shape[0]
    HW = H * W
    lut_words = L * L * L * 3
    lut_pad = -(-lut_words // 16) * 16
    img = image.reshape(B * C * HW)
    lutf = jnp.concatenate(
        [lut.reshape(-1), jnp.zeros((lut_pad - lut_words,), jnp.float32)]
    )
    out = _make_sc_kernel(B, HW, L)(img, lutf)
    return out.reshape(image.shape)
